# baseline (speedup 1.0000x reference)
import functools

import jax
import jax.numpy as jnp
from jax import lax
from jax.experimental import pallas as pl
from jax.experimental.pallas import tpu as pltpu
from jax.experimental.pallas import tpu_sc as plsc

N_NODES = 10000
N_EDGES = 320000
D_IN = 128
D_EDGE = 16
HID = 128
OUT = 16
N_LAYERS = 3
N_GRAPHS = 128
EPS = 1e-7
HALF = HID // 2


def _mlp2_split_body(x_ref, w1_ref, b1_ref, w2_ref, b2_ref, out_ref):
    h = jnp.dot(x_ref[...], w1_ref[...], preferred_element_type=jnp.float32)
    h = jnp.maximum(h + b1_ref[...], 0.0)
    h = jnp.dot(h, w2_ref[...], preferred_element_type=jnp.float32)
    h = jnp.maximum(h + b2_ref[...], 0.0)
    out_ref[0] = h[:, :HALF]
    out_ref[1] = h[:, HALF:]


def _mlp2_body(x_ref, w1_ref, b1_ref, w2_ref, b2_ref, out_ref):
    h = jnp.dot(x_ref[...], w1_ref[...], preferred_element_type=jnp.float32)
    h = jnp.maximum(h + b1_ref[...], 0.0)
    h = jnp.dot(h, w2_ref[...], preferred_element_type=jnp.float32)
    out_ref[...] = jnp.maximum(h + b2_ref[...], 0.0)


def _node_mlp(x, w1, b1, w2, b2):
    return pl.pallas_call(
        _mlp2_body,
        out_shape=jax.ShapeDtypeStruct((N_NODES, HID), jnp.float32),
    )(x, w1, b1.reshape(1, -1), w2, b2.reshape(1, -1))


def _edge_mlp(ea, w1, b1, w2, b2):
    eb = 10000
    return pl.pallas_call(
        _mlp2_split_body,
        grid=(N_EDGES // eb,),
        in_specs=[
            pl.BlockSpec((eb, D_EDGE), lambda i: (i, 0)),
            pl.BlockSpec((D_EDGE, HID), lambda i: (0, 0)),
            pl.BlockSpec((1, HID), lambda i: (0, 0)),
            pl.BlockSpec((HID, HID), lambda i: (0, 0)),
            pl.BlockSpec((1, HID), lambda i: (0, 0)),
        ],
        out_specs=pl.BlockSpec((2, eb, HALF), lambda i: (0, i, 0)),
        out_shape=jax.ShapeDtypeStruct((2, N_EDGES, HALF), jnp.float32),
    )(ea, w1, b1.reshape(1, -1), w2, b2.reshape(1, -1))


def _node_update_body(relu_after, num_ref, den_ref, h_ref, w1_ref, b1_ref,
                      g_ref, be_ref, w2_ref, b2_ref, out_ref):
    numer = jnp.concatenate([num_ref[0], num_ref[1]], axis=1)
    denom = jnp.concatenate([den_ref[0], den_ref[1]], axis=1)
    out = numer / (denom + 1e-16) + h_ref[...]
    p = jnp.dot(out, w1_ref[...], preferred_element_type=jnp.float32) + b1_ref[...]
    mean = jnp.mean(p, axis=0, keepdims=True)
    var = jnp.mean((p - mean) ** 2, axis=0, keepdims=True)
    p = (p - mean) / jnp.sqrt(var + 1e-5) * g_ref[...] + be_ref[...]
    p = jnp.maximum(p, 0.0)
    r = jnp.dot(p, w2_ref[...], preferred_element_type=jnp.float32) + b2_ref[...]
    if relu_after:
        r = jnp.maximum(r, 0.0)
    out_ref[...] = r


def _node_update(num2, den2, h, w1, b1, g, be, w2, b2, relu_after):
    return pl.pallas_call(
        functools.partial(_node_update_body, relu_after),
        out_shape=jax.ShapeDtypeStruct((N_NODES, HID), jnp.float32),
    )(num2, den2, h, w1, b1.reshape(1, -1), g.reshape(1, -1),
      be.reshape(1, -1), w2, b2.reshape(1, -1))


def _pool_body(h_ref, batch_ref, l1w_ref, l1b_ref, l2w_ref, l2b_ref, out_ref):
    h = h_ref[...]
    gids = lax.broadcasted_iota(jnp.int32, (N_GRAPHS, N_NODES), 0)
    onehot = (batch_ref[...] == gids).astype(jnp.float32)
    pooled = jnp.dot(onehot, h, preferred_element_type=jnp.float32)
    z = jnp.dot(pooled, l1w_ref[...], preferred_element_type=jnp.float32)
    z = jnp.maximum(z + l1b_ref[...], 0.0)
    out_ref[...] = jnp.dot(z, l2w_ref[...], preferred_element_type=jnp.float32) + l2b_ref[...]


def _pool_head(h, batch, l1w, l1b, l2w, l2b):
    return pl.pallas_call(
        _pool_body,
        out_shape=jax.ShapeDtypeStruct((N_GRAPHS, OUT), jnp.float32),
    )(h, batch.reshape(1, -1), l1w, l1b.reshape(1, -1), l2w, l2b.reshape(1, -1))


TILES = 16
EPT = N_EDGES // TILES
CH = 80
NCHUNK = EPT // CH
RPT = 624
ZCH = 8

_mesh = plsc.VectorSubcoreMesh(core_axis_name="c", subcore_axis_name="s",
                               num_cores=2)


@functools.partial(
    pl.kernel,
    out_type=(jax.ShapeDtypeStruct((2 * N_NODES, HALF), jnp.float32),
              jax.ShapeDtypeStruct((2 * N_NODES, HALF), jnp.float32)),
    mesh=_mesh,
    compiler_params=pltpu.CompilerParams(use_tc_tiling_on_sc=False),
    scratch_types=(
        pltpu.VMEM((CH,), jnp.int32),
        pltpu.VMEM((CH,), jnp.int32),
        pltpu.VMEM((CH, HID), jnp.float32),
        pltpu.VMEM((CH, HALF), jnp.float32),
        pltpu.VMEM((CH, HALF), jnp.float32),
        pltpu.VMEM((CH, HALF), jnp.float32),
        pltpu.VMEM((ZCH, HALF), jnp.float32),
        pltpu.VMEM((16,), jnp.float32),
        pltpu.VMEM_SHARED((N_NODES, HALF), jnp.float32),
        pltpu.VMEM_SHARED((N_NODES, HALF), jnp.float32),
        pltpu.SemaphoreType.DMA,
    ),
)
def _edge_kernel(h_hbm, e_hbm, sd_hbm, t_hbm, num_hbm, den_hbm,
                 gidx, didx, hrows, erows, mwrows, wrows, zbuf, tv,
                 accn, accd, sem):
    c = lax.axis_index("c")
    s = lax.axis_index("s")

    def zfill(r, carry):
        for v in range(HALF // 16):
            zbuf[r, pl.ds(v * 16, 16)] = jnp.zeros((16,), jnp.float32)
        return carry
    lax.fori_loop(0, ZCH, zfill, 0)
    row0 = s * RPT

    def zcopy(z, carry):
        pltpu.sync_copy(zbuf, accn.at[pl.ds(row0 + z * ZCH, ZCH)])
        pltpu.sync_copy(zbuf, accd.at[pl.ds(row0 + z * ZCH, ZCH)])
        return carry
    lax.fori_loop(0, RPT // ZCH, zcopy, 0)

    @pl.when(s == TILES - 1)
    def _zero_tail():
        for z in range(2):
            pltpu.sync_copy(zbuf, accn.at[pl.ds(TILES * RPT + z * ZCH, ZCH)])
            pltpu.sync_copy(zbuf, accd.at[pl.ds(TILES * RPT + z * ZCH, ZCH)])

    pltpu.sync_copy(t_hbm, tv)
    plsc.subcore_barrier()

    cN = c * N_NODES
    col0 = c * HALF
    ebase = s * EPT

    def chunk(j, carry):
        base = ebase + j * CH
        pltpu.sync_copy(sd_hbm.at[pl.ds(base, CH)], gidx)
        pltpu.sync_copy(sd_hbm.at[pl.ds(N_EDGES + base, CH)], didx)
        pltpu.async_copy(h_hbm.at[gidx], hrows, sem).wait()
        pltpu.sync_copy(e_hbm.at[pl.ds(c * N_EDGES + base, CH)], erows)
        tvec = tv[...]

        def row(r, rcarry):
            for v in range(HALF // 16):
                sl = pl.ds(v * 16, 16)
                m = jnp.maximum(hrows[r, pl.ds(col0 + v * 16, 16)]
                                + erows[r, sl], 0.0) + EPS
                w = jnp.exp(tvec * m)
                mwrows[r, sl] = m * w
                wrows[r, sl] = w
            return rcarry
        lax.fori_loop(0, CH, row, 0, unroll=8)
        pltpu.sync_copy(mwrows, accn.at[didx], add=True)
        pltpu.sync_copy(wrows, accd.at[didx], add=True)
        return carry
    lax.fori_loop(0, NCHUNK, chunk, 0)

    plsc.subcore_barrier()

    def wcopy(z, carry):
        r = row0 + z * ZCH
        pltpu.sync_copy(accn.at[pl.ds(r, ZCH)], zbuf)
        pltpu.sync_copy(zbuf, num_hbm.at[pl.ds(cN + r, ZCH)])
        pltpu.sync_copy(accd.at[pl.ds(r, ZCH)], zbuf)
        pltpu.sync_copy(zbuf, den_hbm.at[pl.ds(cN + r, ZCH)])
        return carry
    lax.fori_loop(0, RPT // ZCH, wcopy, 0)

    @pl.when(s == TILES - 1)
    def _write_tail():
        for z in range(2):
            r = TILES * RPT + z * ZCH
            pltpu.sync_copy(accn.at[pl.ds(r, ZCH)], zbuf)
            pltpu.sync_copy(zbuf, num_hbm.at[pl.ds(cN + r, ZCH)])
            pltpu.sync_copy(accd.at[pl.ds(r, ZCH)], zbuf)
            pltpu.sync_copy(zbuf, den_hbm.at[pl.ds(cN + r, ZCH)])


def _edge_pass(h, e2, sd_flat, t_i):
    num_flat, den_flat = _edge_kernel(
        h,
        e2.reshape(2 * N_EDGES, HALF),
        sd_flat,
        jnp.full((16,), t_i, jnp.float32))
    return (num_flat.reshape(2, N_NODES, HALF),
            den_flat.reshape(2, N_NODES, HALF))


def kernel(x, edge_attr, aW1, ab1, aW2, ab2, bW1, bb1, bW2, bb2, t,
           cW1, cb1, cg, cbe, cW2, cb2, l1W, l1b, l2W, l2b, edge_index, batch):
    sd_flat = edge_index.reshape(-1)
    h = _node_mlp(x, aW1, ab1, aW2, ab2)
    e2 = _edge_mlp(edge_attr, bW1, bb1, bW2, bb2)
    for i in range(N_LAYERS):
        num2, den2 = _edge_pass(h, e2, sd_flat, t[i])
        h = _node_update(num2, den2, h, cW1[i], cb1[i], cg[i], cbe[i],
                         cW2[i], cb2[i], relu_after=(i < N_LAYERS - 1))
    return _pool_head(h, batch, l1W, l1b, l2W, l2b)


# speedup vs baseline: 1.9952x; 1.9952x over previous
import functools

import jax
import jax.numpy as jnp
from jax import lax
from jax.experimental import pallas as pl
from jax.experimental.pallas import tpu as pltpu
from jax.experimental.pallas import tpu_sc as plsc

N_NODES = 10000
N_EDGES = 320000
D_IN = 128
D_EDGE = 16
HID = 128
OUT = 16
N_LAYERS = 3
N_GRAPHS = 128
EPS = 1e-7
HALF = HID // 2


def _mlp2_split_body(x_ref, w1_ref, b1_ref, w2_ref, b2_ref, out_ref):
    h = jnp.dot(x_ref[...], w1_ref[...], preferred_element_type=jnp.float32)
    h = jnp.maximum(h + b1_ref[...], 0.0)
    h = jnp.dot(h, w2_ref[...], preferred_element_type=jnp.float32)
    h = jnp.maximum(h + b2_ref[...], 0.0)
    out_ref[0] = h[:, :HALF]
    out_ref[1] = h[:, HALF:]


def _mlp2_body(x_ref, w1_ref, b1_ref, w2_ref, b2_ref, out_ref):
    h = jnp.dot(x_ref[...], w1_ref[...], preferred_element_type=jnp.float32)
    h = jnp.maximum(h + b1_ref[...], 0.0)
    h = jnp.dot(h, w2_ref[...], preferred_element_type=jnp.float32)
    out_ref[...] = jnp.maximum(h + b2_ref[...], 0.0)


def _node_mlp(x, w1, b1, w2, b2):
    return pl.pallas_call(
        _mlp2_body,
        out_shape=jax.ShapeDtypeStruct((N_NODES, HID), jnp.float32),
    )(x, w1, b1.reshape(1, -1), w2, b2.reshape(1, -1))


def _edge_mlp(ea, w1, b1, w2, b2):
    eb = 10000
    return pl.pallas_call(
        _mlp2_split_body,
        grid=(N_EDGES // eb,),
        in_specs=[
            pl.BlockSpec((eb, D_EDGE), lambda i: (i, 0)),
            pl.BlockSpec((D_EDGE, HID), lambda i: (0, 0)),
            pl.BlockSpec((1, HID), lambda i: (0, 0)),
            pl.BlockSpec((HID, HID), lambda i: (0, 0)),
            pl.BlockSpec((1, HID), lambda i: (0, 0)),
        ],
        out_specs=pl.BlockSpec((2, eb, HALF), lambda i: (0, i, 0)),
        out_shape=jax.ShapeDtypeStruct((2, N_EDGES, HALF), jnp.float32),
    )(ea, w1, b1.reshape(1, -1), w2, b2.reshape(1, -1))


def _node_update_body(relu_after, num_ref, den_ref, h_ref, w1_ref, b1_ref,
                      g_ref, be_ref, w2_ref, b2_ref, out_ref):
    numer = jnp.concatenate([num_ref[0], num_ref[1]], axis=1)
    denom = jnp.concatenate([den_ref[0], den_ref[1]], axis=1)
    out = numer / (denom + 1e-16) + h_ref[...]
    p = jnp.dot(out, w1_ref[...], preferred_element_type=jnp.float32) + b1_ref[...]
    mean = jnp.mean(p, axis=0, keepdims=True)
    var = jnp.mean((p - mean) ** 2, axis=0, keepdims=True)
    p = (p - mean) / jnp.sqrt(var + 1e-5) * g_ref[...] + be_ref[...]
    p = jnp.maximum(p, 0.0)
    r = jnp.dot(p, w2_ref[...], preferred_element_type=jnp.float32) + b2_ref[...]
    if relu_after:
        r = jnp.maximum(r, 0.0)
    out_ref[...] = r


def _node_update(num2, den2, h, w1, b1, g, be, w2, b2, relu_after):
    return pl.pallas_call(
        functools.partial(_node_update_body, relu_after),
        out_shape=jax.ShapeDtypeStruct((N_NODES, HID), jnp.float32),
    )(num2, den2, h, w1, b1.reshape(1, -1), g.reshape(1, -1),
      be.reshape(1, -1), w2, b2.reshape(1, -1))


def _pool_body(h_ref, batch_ref, l1w_ref, l1b_ref, l2w_ref, l2b_ref, out_ref):
    h = h_ref[...]
    gids = lax.broadcasted_iota(jnp.int32, (N_GRAPHS, N_NODES), 0)
    onehot = (batch_ref[...] == gids).astype(jnp.float32)
    pooled = jnp.dot(onehot, h, preferred_element_type=jnp.float32)
    z = jnp.dot(pooled, l1w_ref[...], preferred_element_type=jnp.float32)
    z = jnp.maximum(z + l1b_ref[...], 0.0)
    out_ref[...] = jnp.dot(z, l2w_ref[...], preferred_element_type=jnp.float32) + l2b_ref[...]


def _pool_head(h, batch, l1w, l1b, l2w, l2b):
    return pl.pallas_call(
        _pool_body,
        out_shape=jax.ShapeDtypeStruct((N_GRAPHS, OUT), jnp.float32),
    )(h, batch.reshape(1, -1), l1w, l1b.reshape(1, -1), l2w, l2b.reshape(1, -1))


TILES = 16
EPT = N_EDGES // TILES
CH = 80
NCHUNK = EPT // CH
RPT = 624
ZCH = 8

_mesh = plsc.VectorSubcoreMesh(core_axis_name="c", subcore_axis_name="s",
                               num_cores=2)


@functools.partial(
    pl.kernel,
    out_type=(jax.ShapeDtypeStruct((2 * N_NODES, HALF), jnp.float32),
              jax.ShapeDtypeStruct((2 * N_NODES, HALF), jnp.float32)),
    mesh=_mesh,
    compiler_params=pltpu.CompilerParams(use_tc_tiling_on_sc=False),
    scratch_types=(
        pltpu.VMEM((CH,), jnp.int32),
        pltpu.VMEM((CH,), jnp.int32),
        pltpu.VMEM((CH, HID), jnp.float32),
        pltpu.VMEM((CH, HALF), jnp.float32),
        pltpu.VMEM((CH, HALF), jnp.float32),
        pltpu.VMEM((CH, HALF), jnp.float32),
        pltpu.VMEM((ZCH, HALF), jnp.float32),
        pltpu.VMEM((16,), jnp.float32),
        pltpu.VMEM_SHARED((N_NODES, HALF), jnp.float32),
        pltpu.VMEM_SHARED((N_NODES, HALF), jnp.float32),
        pltpu.SemaphoreType.DMA,
    ),
)
def _edge_kernel(h_hbm, e_hbm, sd_hbm, t_hbm, num_hbm, den_hbm,
                 gidx, didx, hrows, erows, mwrows, wrows, zbuf, tv,
                 accn, accd, sem):
    c = lax.axis_index("c")
    s = lax.axis_index("s")

    def zfill(r, carry):
        for v in range(HALF // 16):
            zbuf[r, pl.ds(v * 16, 16)] = jnp.zeros((16,), jnp.float32)
        return carry
    lax.fori_loop(0, ZCH, zfill, 0)
    row0 = s * RPT

    def zcopy(z, carry):
        pltpu.sync_copy(zbuf, accn.at[pl.ds(row0 + z * ZCH, ZCH)])
        pltpu.sync_copy(zbuf, accd.at[pl.ds(row0 + z * ZCH, ZCH)])
        return carry
    lax.fori_loop(0, RPT // ZCH, zcopy, 0)

    @pl.when(s == TILES - 1)
    def _zero_tail():
        for z in range(2):
            pltpu.sync_copy(zbuf, accn.at[pl.ds(TILES * RPT + z * ZCH, ZCH)])
            pltpu.sync_copy(zbuf, accd.at[pl.ds(TILES * RPT + z * ZCH, ZCH)])

    pltpu.sync_copy(t_hbm, tv)
    plsc.subcore_barrier()

    cN = c * N_NODES
    col0 = c * HALF
    ebase = s * EPT

    def chunk(j, carry):
        base = ebase + j * CH
        pltpu.sync_copy(sd_hbm.at[pl.ds(base, CH)], gidx)
        pltpu.sync_copy(sd_hbm.at[pl.ds(N_EDGES + base, CH)], didx)
        pltpu.async_copy(h_hbm.at[gidx], hrows, sem).wait()
        pltpu.sync_copy(e_hbm.at[pl.ds(c * N_EDGES + base, CH)], erows)
        tvec = tv[...]

        @plsc.parallel_loop(0, CH, unroll=4)
        def row(r):
            for v in range(HALF // 16):
                sl = pl.ds(v * 16, 16)
                m = jnp.maximum(hrows[r, pl.ds(col0 + v * 16, 16)]
                                + erows[r, sl], 0.0) + EPS
                w = jnp.exp(tvec * m)
                mwrows[r, sl] = m * w
                wrows[r, sl] = w
        pltpu.sync_copy(mwrows, accn.at[didx], add=True)
        pltpu.sync_copy(wrows, accd.at[didx], add=True)
        return carry
    lax.fori_loop(0, NCHUNK, chunk, 0)

    plsc.subcore_barrier()

    def wcopy(z, carry):
        r = row0 + z * ZCH
        pltpu.sync_copy(accn.at[pl.ds(r, ZCH)], zbuf)
        pltpu.sync_copy(zbuf, num_hbm.at[pl.ds(cN + r, ZCH)])
        pltpu.sync_copy(accd.at[pl.ds(r, ZCH)], zbuf)
        pltpu.sync_copy(zbuf, den_hbm.at[pl.ds(cN + r, ZCH)])
        return carry
    lax.fori_loop(0, RPT // ZCH, wcopy, 0)

    @pl.when(s == TILES - 1)
    def _write_tail():
        for z in range(2):
            r = TILES * RPT + z * ZCH
            pltpu.sync_copy(accn.at[pl.ds(r, ZCH)], zbuf)
            pltpu.sync_copy(zbuf, num_hbm.at[pl.ds(cN + r, ZCH)])
            pltpu.sync_copy(accd.at[pl.ds(r, ZCH)], zbuf)
            pltpu.sync_copy(zbuf, den_hbm.at[pl.ds(cN + r, ZCH)])


def _edge_pass(h, e2, sd_flat, t_i):
    num_flat, den_flat = _edge_kernel(
        h,
        e2.reshape(2 * N_EDGES, HALF),
        sd_flat,
        jnp.full((16,), t_i, jnp.float32))
    return (num_flat.reshape(2, N_NODES, HALF),
            den_flat.reshape(2, N_NODES, HALF))


def kernel(x, edge_attr, aW1, ab1, aW2, ab2, bW1, bb1, bW2, bb2, t,
           cW1, cb1, cg, cbe, cW2, cb2, l1W, l1b, l2W, l2b, edge_index, batch):
    sd_flat = edge_index.reshape(-1)
    h = _node_mlp(x, aW1, ab1, aW2, ab2)
    e2 = _edge_mlp(edge_attr, bW1, bb1, bW2, bb2)
    for i in range(N_LAYERS):
        num2, den2 = _edge_pass(h, e2, sd_flat, t[i])
        h = _node_update(num2, den2, h, cW1[i], cb1[i], cg[i], cbe[i],
                         cW2[i], cb2[i], relu_after=(i < N_LAYERS - 1))
    return _pool_head(h, batch, l1W, l1b, l2W, l2b)


# speedup vs baseline: 3.0452x; 1.5263x over previous
import functools

import jax
import jax.numpy as jnp
from jax import lax
from jax.experimental import pallas as pl
from jax.experimental.pallas import tpu as pltpu
from jax.experimental.pallas import tpu_sc as plsc

N_NODES = 10000
N_EDGES = 320000
D_IN = 128
D_EDGE = 16
HID = 128
OUT = 16
N_LAYERS = 3
N_GRAPHS = 128
EPS = 1e-7
HALF = HID // 2


def _mlp2_split_body(x_ref, w1_ref, b1_ref, w2_ref, b2_ref, out_ref):
    h = jnp.dot(x_ref[...], w1_ref[...], preferred_element_type=jnp.float32)
    h = jnp.maximum(h + b1_ref[...], 0.0)
    h = jnp.dot(h, w2_ref[...], preferred_element_type=jnp.float32)
    h = jnp.maximum(h + b2_ref[...], 0.0)
    out_ref[0] = h[:, :HALF]
    out_ref[1] = h[:, HALF:]


def _mlp2_body(x_ref, w1_ref, b1_ref, w2_ref, b2_ref, out_ref):
    h = jnp.dot(x_ref[...], w1_ref[...], preferred_element_type=jnp.float32)
    h = jnp.maximum(h + b1_ref[...], 0.0)
    h = jnp.dot(h, w2_ref[...], preferred_element_type=jnp.float32)
    out_ref[...] = jnp.maximum(h + b2_ref[...], 0.0)


def _node_mlp(x, w1, b1, w2, b2):
    return pl.pallas_call(
        _mlp2_body,
        out_shape=jax.ShapeDtypeStruct((N_NODES, HID), jnp.float32),
    )(x, w1, b1.reshape(1, -1), w2, b2.reshape(1, -1))


def _edge_mlp(ea, w1, b1, w2, b2):
    eb = 10000
    return pl.pallas_call(
        _mlp2_split_body,
        grid=(N_EDGES // eb,),
        in_specs=[
            pl.BlockSpec((eb, D_EDGE), lambda i: (i, 0)),
            pl.BlockSpec((D_EDGE, HID), lambda i: (0, 0)),
            pl.BlockSpec((1, HID), lambda i: (0, 0)),
            pl.BlockSpec((HID, HID), lambda i: (0, 0)),
            pl.BlockSpec((1, HID), lambda i: (0, 0)),
        ],
        out_specs=pl.BlockSpec((2, eb, HALF), lambda i: (0, i, 0)),
        out_shape=jax.ShapeDtypeStruct((2, N_EDGES, HALF), jnp.float32),
    )(ea, w1, b1.reshape(1, -1), w2, b2.reshape(1, -1))


def _node_update_body(relu_after, num_ref, den_ref, h_ref, w1_ref, b1_ref,
                      g_ref, be_ref, w2_ref, b2_ref, out_ref):
    numer = jnp.concatenate([num_ref[0], num_ref[1]], axis=1)
    denom = jnp.concatenate([den_ref[0], den_ref[1]], axis=1)
    out = numer / (denom + 1e-16) + h_ref[...]
    p = jnp.dot(out, w1_ref[...], preferred_element_type=jnp.float32) + b1_ref[...]
    mean = jnp.mean(p, axis=0, keepdims=True)
    var = jnp.mean((p - mean) ** 2, axis=0, keepdims=True)
    p = (p - mean) / jnp.sqrt(var + 1e-5) * g_ref[...] + be_ref[...]
    p = jnp.maximum(p, 0.0)
    r = jnp.dot(p, w2_ref[...], preferred_element_type=jnp.float32) + b2_ref[...]
    if relu_after:
        r = jnp.maximum(r, 0.0)
    out_ref[...] = r


def _node_update(num2, den2, h, w1, b1, g, be, w2, b2, relu_after):
    return pl.pallas_call(
        functools.partial(_node_update_body, relu_after),
        out_shape=jax.ShapeDtypeStruct((N_NODES, HID), jnp.float32),
    )(num2, den2, h, w1, b1.reshape(1, -1), g.reshape(1, -1),
      be.reshape(1, -1), w2, b2.reshape(1, -1))


def _pool_body(h_ref, batch_ref, l1w_ref, l1b_ref, l2w_ref, l2b_ref, out_ref):
    h = h_ref[...]
    gids = lax.broadcasted_iota(jnp.int32, (N_GRAPHS, N_NODES), 0)
    onehot = (batch_ref[...] == gids).astype(jnp.float32)
    pooled = jnp.dot(onehot, h, preferred_element_type=jnp.float32)
    z = jnp.dot(pooled, l1w_ref[...], preferred_element_type=jnp.float32)
    z = jnp.maximum(z + l1b_ref[...], 0.0)
    out_ref[...] = jnp.dot(z, l2w_ref[...], preferred_element_type=jnp.float32) + l2b_ref[...]


def _pool_head(h, batch, l1w, l1b, l2w, l2b):
    return pl.pallas_call(
        _pool_body,
        out_shape=jax.ShapeDtypeStruct((N_GRAPHS, OUT), jnp.float32),
    )(h, batch.reshape(1, -1), l1w, l1b.reshape(1, -1), l2w, l2b.reshape(1, -1))


TILES = 16
EPT = N_EDGES // TILES
CH = 40
NCHUNK = EPT // CH
RPT = 624
ZCH = 8

_mesh = plsc.VectorSubcoreMesh(core_axis_name="c", subcore_axis_name="s",
                               num_cores=2)


@functools.partial(
    pl.kernel,
    out_type=(jax.ShapeDtypeStruct((2 * N_NODES, HALF), jnp.float32),
              jax.ShapeDtypeStruct((2 * N_NODES, HALF), jnp.float32)),
    mesh=_mesh,
    compiler_params=pltpu.CompilerParams(use_tc_tiling_on_sc=False),
    scratch_types=(
        pltpu.VMEM((CH,), jnp.int32),
        pltpu.VMEM((CH,), jnp.int32),
        pltpu.VMEM((CH,), jnp.int32),
        pltpu.VMEM((CH,), jnp.int32),
        pltpu.VMEM((CH, HID), jnp.float32),
        pltpu.VMEM((CH, HID), jnp.float32),
        pltpu.VMEM((CH, HALF), jnp.float32),
        pltpu.VMEM((CH, HALF), jnp.float32),
        pltpu.VMEM((CH, HALF), jnp.float32),
        pltpu.VMEM((CH, HALF), jnp.float32),
        pltpu.VMEM((ZCH, HALF), jnp.float32),
        pltpu.VMEM((16,), jnp.float32),
        pltpu.VMEM_SHARED((N_NODES, HALF), jnp.float32),
        pltpu.VMEM_SHARED((N_NODES, HALF), jnp.float32),
        pltpu.SemaphoreType.DMA,
        pltpu.SemaphoreType.DMA,
        pltpu.SemaphoreType.DMA,
        pltpu.SemaphoreType.DMA,
        pltpu.SemaphoreType.DMA,
        pltpu.SemaphoreType.DMA,
    ),
)
def _edge_kernel(h_hbm, e_hbm, sd_hbm, t_hbm, num_hbm, den_hbm,
                 gidxA, gidxB, didxA, didxB, hrowsA, hrowsB, erowsA, erowsB,
                 mwrows, wrows, zbuf, tv, accn, accd,
                 ssA, ssB, gsA, gsB, esA, esB):
    c = lax.axis_index("c")
    s = lax.axis_index("s")

    def zfill(r, carry):
        for v in range(HALF // 16):
            zbuf[r, pl.ds(v * 16, 16)] = jnp.zeros((16,), jnp.float32)
        return carry
    lax.fori_loop(0, ZCH, zfill, 0)
    row0 = s * RPT

    def zcopy(z, carry):
        pltpu.sync_copy(zbuf, accn.at[pl.ds(row0 + z * ZCH, ZCH)])
        pltpu.sync_copy(zbuf, accd.at[pl.ds(row0 + z * ZCH, ZCH)])
        return carry
    lax.fori_loop(0, RPT // ZCH, zcopy, 0)

    @pl.when(s == TILES - 1)
    def _zero_tail():
        for z in range(2):
            pltpu.sync_copy(zbuf, accn.at[pl.ds(TILES * RPT + z * ZCH, ZCH)])
            pltpu.sync_copy(zbuf, accd.at[pl.ds(TILES * RPT + z * ZCH, ZCH)])

    pltpu.sync_copy(t_hbm, tv)
    plsc.subcore_barrier()

    cN = c * N_NODES
    col0 = c * HALF
    ebase = s * EPT

    def issue_sd(j, gx, dx, sm):
        base = ebase + j * CH
        pltpu.async_copy(sd_hbm.at[pl.ds(base, CH)], gx, sm)
        pltpu.async_copy(sd_hbm.at[pl.ds(N_EDGES + base, CH)], dx, sm)

    def wait_sd(gx, dx, sm):
        pltpu.make_async_copy(sd_hbm.at[pl.ds(0, CH)], gx, sm).wait()
        pltpu.make_async_copy(sd_hbm.at[pl.ds(0, CH)], dx, sm).wait()

    def issue_ge(j, gx, hr, er, gs, es):
        base = ebase + j * CH
        pltpu.async_copy(h_hbm.at[gx], hr, gs)
        pltpu.async_copy(e_hbm.at[pl.ds(c * N_EDGES + base, CH)], er, es)

    def wait_ge(gx, hr, er, gs, es):
        pltpu.make_async_copy(h_hbm.at[gx], hr, gs).wait()
        pltpu.make_async_copy(e_hbm.at[pl.ds(0, CH)], er, es).wait()

    def compute_scatter(hr, er, dx):
        tvec = tv[...]

        @plsc.parallel_loop(0, CH, unroll=4)
        def row(r):
            for v in range(HALF // 16):
                sl = pl.ds(v * 16, 16)
                m = jnp.maximum(hr[r, pl.ds(col0 + v * 16, 16)]
                                + er[r, sl], 0.0) + EPS
                w = jnp.exp(tvec * m)
                mwrows[r, sl] = m * w
                wrows[r, sl] = w
        pltpu.sync_copy(mwrows, accn.at[dx], add=True)
        pltpu.sync_copy(wrows, accd.at[dx], add=True)

    issue_sd(0, gidxA, didxA, ssA)
    wait_sd(gidxA, didxA, ssA)
    issue_ge(0, gidxA, hrowsA, erowsA, gsA, esA)
    issue_sd(1, gidxB, didxB, ssB)

    @pl.loop(0, NCHUNK // 2 - 1)
    def piter(k):
        j = 2 * k
        wait_sd(gidxB, didxB, ssB)
        issue_ge(j + 1, gidxB, hrowsB, erowsB, gsB, esB)
        wait_ge(gidxA, hrowsA, erowsA, gsA, esA)
        compute_scatter(hrowsA, erowsA, didxA)
        issue_sd(j + 2, gidxA, didxA, ssA)
        wait_ge(gidxB, hrowsB, erowsB, gsB, esB)
        compute_scatter(hrowsB, erowsB, didxB)
        issue_sd(j + 3, gidxB, didxB, ssB)
        wait_sd(gidxA, didxA, ssA)
        issue_ge(j + 2, gidxA, hrowsA, erowsA, gsA, esA)

    wait_sd(gidxB, didxB, ssB)
    issue_ge(NCHUNK - 1, gidxB, hrowsB, erowsB, gsB, esB)
    wait_ge(gidxA, hrowsA, erowsA, gsA, esA)
    compute_scatter(hrowsA, erowsA, didxA)
    wait_ge(gidxB, hrowsB, erowsB, gsB, esB)
    compute_scatter(hrowsB, erowsB, didxB)

    plsc.subcore_barrier()

    def wcopy(z, carry):
        r = row0 + z * ZCH
        pltpu.sync_copy(accn.at[pl.ds(r, ZCH)], zbuf)
        pltpu.sync_copy(zbuf, num_hbm.at[pl.ds(cN + r, ZCH)])
        pltpu.sync_copy(accd.at[pl.ds(r, ZCH)], zbuf)
        pltpu.sync_copy(zbuf, den_hbm.at[pl.ds(cN + r, ZCH)])
        return carry
    lax.fori_loop(0, RPT // ZCH, wcopy, 0)

    @pl.when(s == TILES - 1)
    def _write_tail():
        for z in range(2):
            r = TILES * RPT + z * ZCH
            pltpu.sync_copy(accn.at[pl.ds(r, ZCH)], zbuf)
            pltpu.sync_copy(zbuf, num_hbm.at[pl.ds(cN + r, ZCH)])
            pltpu.sync_copy(accd.at[pl.ds(r, ZCH)], zbuf)
            pltpu.sync_copy(zbuf, den_hbm.at[pl.ds(cN + r, ZCH)])


def _edge_pass(h, e2, sd_flat, t_i):
    num_flat, den_flat = _edge_kernel(
        h,
        e2.reshape(2 * N_EDGES, HALF),
        sd_flat,
        jnp.full((16,), t_i, jnp.float32))
    return (num_flat.reshape(2, N_NODES, HALF),
            den_flat.reshape(2, N_NODES, HALF))


def kernel(x, edge_attr, aW1, ab1, aW2, ab2, bW1, bb1, bW2, bb2, t,
           cW1, cb1, cg, cbe, cW2, cb2, l1W, l1b, l2W, l2b, edge_index, batch):
    sd_flat = edge_index.reshape(-1)
    h = _node_mlp(x, aW1, ab1, aW2, ab2)
    e2 = _edge_mlp(edge_attr, bW1, bb1, bW2, bb2)
    for i in range(N_LAYERS):
        num2, den2 = _edge_pass(h, e2, sd_flat, t[i])
        h = _node_update(num2, den2, h, cW1[i], cb1[i], cg[i], cbe[i],
                         cW2[i], cb2[i], relu_after=(i < N_LAYERS - 1))
    return _pool_head(h, batch, l1W, l1b, l2W, l2b)


# speedup vs baseline: 3.1534x; 1.0355x over previous
import functools

import jax
import jax.numpy as jnp
from jax import lax
from jax.experimental import pallas as pl
from jax.experimental.pallas import tpu as pltpu
from jax.experimental.pallas import tpu_sc as plsc

N_NODES = 10000
N_EDGES = 320000
D_IN = 128
D_EDGE = 16
HID = 128
OUT = 16
N_LAYERS = 3
N_GRAPHS = 128
EPS = 1e-7
HALF = HID // 2


def _mlp2_split_body(x_ref, w1_ref, b1_ref, w2_ref, b2_ref, out_ref):
    h = jnp.dot(x_ref[...], w1_ref[...], preferred_element_type=jnp.float32)
    h = jnp.maximum(h + b1_ref[...], 0.0)
    h = jnp.dot(h, w2_ref[...], preferred_element_type=jnp.float32)
    h = jnp.maximum(h + b2_ref[...], 0.0)
    out_ref[0] = h[:, :HALF]
    out_ref[1] = h[:, HALF:]


def _mlp2_body(x_ref, w1_ref, b1_ref, w2_ref, b2_ref, out_ref):
    h = jnp.dot(x_ref[...], w1_ref[...], preferred_element_type=jnp.float32)
    h = jnp.maximum(h + b1_ref[...], 0.0)
    h = jnp.dot(h, w2_ref[...], preferred_element_type=jnp.float32)
    out_ref[...] = jnp.maximum(h + b2_ref[...], 0.0)


def _node_mlp(x, w1, b1, w2, b2):
    return pl.pallas_call(
        _mlp2_body,
        out_shape=jax.ShapeDtypeStruct((N_NODES, HID), jnp.float32),
    )(x, w1, b1.reshape(1, -1), w2, b2.reshape(1, -1))


def _edge_mlp(ea, w1, b1, w2, b2):
    eb = 10000
    return pl.pallas_call(
        _mlp2_split_body,
        grid=(N_EDGES // eb,),
        in_specs=[
            pl.BlockSpec((eb, D_EDGE), lambda i: (i, 0)),
            pl.BlockSpec((D_EDGE, HID), lambda i: (0, 0)),
            pl.BlockSpec((1, HID), lambda i: (0, 0)),
            pl.BlockSpec((HID, HID), lambda i: (0, 0)),
            pl.BlockSpec((1, HID), lambda i: (0, 0)),
        ],
        out_specs=pl.BlockSpec((2, eb, HALF), lambda i: (0, i, 0)),
        out_shape=jax.ShapeDtypeStruct((2, N_EDGES, HALF), jnp.float32),
    )(ea, w1, b1.reshape(1, -1), w2, b2.reshape(1, -1))


def _node_update_body(relu_after, num_ref, den_ref, h_ref, w1_ref, b1_ref,
                      g_ref, be_ref, w2_ref, b2_ref, out_ref):
    numer = jnp.concatenate([num_ref[0], num_ref[1]], axis=1)
    denom = jnp.concatenate([den_ref[0], den_ref[1]], axis=1)
    out = numer / (denom + 1e-16) + h_ref[...]
    p = jnp.dot(out, w1_ref[...], preferred_element_type=jnp.float32) + b1_ref[...]
    mean = jnp.mean(p, axis=0, keepdims=True)
    var = jnp.mean((p - mean) ** 2, axis=0, keepdims=True)
    p = (p - mean) / jnp.sqrt(var + 1e-5) * g_ref[...] + be_ref[...]
    p = jnp.maximum(p, 0.0)
    r = jnp.dot(p, w2_ref[...], preferred_element_type=jnp.float32) + b2_ref[...]
    if relu_after:
        r = jnp.maximum(r, 0.0)
    out_ref[...] = r


def _node_update(num2, den2, h, w1, b1, g, be, w2, b2, relu_after):
    return pl.pallas_call(
        functools.partial(_node_update_body, relu_after),
        out_shape=jax.ShapeDtypeStruct((N_NODES, HID), jnp.float32),
    )(num2, den2, h, w1, b1.reshape(1, -1), g.reshape(1, -1),
      be.reshape(1, -1), w2, b2.reshape(1, -1))


def _pool_body(h_ref, batch_ref, l1w_ref, l1b_ref, l2w_ref, l2b_ref, out_ref):
    h = h_ref[...]
    gids = lax.broadcasted_iota(jnp.int32, (N_GRAPHS, N_NODES), 0)
    onehot = (batch_ref[...] == gids).astype(jnp.float32)
    pooled = jnp.dot(onehot, h, preferred_element_type=jnp.float32)
    z = jnp.dot(pooled, l1w_ref[...], preferred_element_type=jnp.float32)
    z = jnp.maximum(z + l1b_ref[...], 0.0)
    out_ref[...] = jnp.dot(z, l2w_ref[...], preferred_element_type=jnp.float32) + l2b_ref[...]


def _pool_head(h, batch, l1w, l1b, l2w, l2b):
    return pl.pallas_call(
        _pool_body,
        out_shape=jax.ShapeDtypeStruct((N_GRAPHS, OUT), jnp.float32),
    )(h, batch.reshape(1, -1), l1w, l1b.reshape(1, -1), l2w, l2b.reshape(1, -1))


TILES = 16
EPT = N_EDGES // TILES
CH = 40
NCHUNK = EPT // CH
RPT = 624
ZCH = 8

_mesh = plsc.VectorSubcoreMesh(core_axis_name="c", subcore_axis_name="s",
                               num_cores=2)


@functools.partial(
    pl.kernel,
    out_type=(jax.ShapeDtypeStruct((2 * N_NODES, HALF), jnp.float32),
              jax.ShapeDtypeStruct((2 * N_NODES, HALF), jnp.float32)),
    mesh=_mesh,
    compiler_params=pltpu.CompilerParams(use_tc_tiling_on_sc=False),
    scratch_types=(
        pltpu.VMEM((CH,), jnp.int32),
        pltpu.VMEM((CH,), jnp.int32),
        pltpu.VMEM((CH,), jnp.int32),
        pltpu.VMEM((CH,), jnp.int32),
        pltpu.VMEM((CH, HID), jnp.float32),
        pltpu.VMEM((CH, HID), jnp.float32),
        pltpu.VMEM((CH, HALF), jnp.float32),
        pltpu.VMEM((CH, HALF), jnp.float32),
        pltpu.VMEM((CH, HALF), jnp.float32),
        pltpu.VMEM((CH, HALF), jnp.float32),
        pltpu.VMEM((ZCH, HALF), jnp.float32),
        pltpu.VMEM((16,), jnp.float32),
        pltpu.VMEM_SHARED((N_NODES, HALF), jnp.float32),
        pltpu.VMEM_SHARED((N_NODES, HALF), jnp.float32),
        pltpu.SemaphoreType.DMA,
        pltpu.SemaphoreType.DMA,
        pltpu.SemaphoreType.DMA,
        pltpu.SemaphoreType.DMA,
        pltpu.SemaphoreType.DMA,
        pltpu.SemaphoreType.DMA,
        pltpu.SemaphoreType.DMA,
    ),
)
def _edge_kernel(h_hbm, e_hbm, sd_hbm, t_hbm, num_hbm, den_hbm,
                 gidxA, gidxB, didxA, didxB, hrowsA, hrowsB, erowsA, erowsB,
                 mwrows, wrows, zbuf, tv, accn, accd,
                 ssA, ssB, gsA, gsB, esA, esB, scS):
    c = lax.axis_index("c")
    s = lax.axis_index("s")

    def zfill(r, carry):
        for v in range(HALF // 16):
            zbuf[r, pl.ds(v * 16, 16)] = jnp.zeros((16,), jnp.float32)
        return carry
    lax.fori_loop(0, ZCH, zfill, 0)
    row0 = s * RPT

    def zcopy(z, carry):
        pltpu.sync_copy(zbuf, accn.at[pl.ds(row0 + z * ZCH, ZCH)])
        pltpu.sync_copy(zbuf, accd.at[pl.ds(row0 + z * ZCH, ZCH)])
        return carry
    lax.fori_loop(0, RPT // ZCH, zcopy, 0)

    @pl.when(s == TILES - 1)
    def _zero_tail():
        for z in range(2):
            pltpu.sync_copy(zbuf, accn.at[pl.ds(TILES * RPT + z * ZCH, ZCH)])
            pltpu.sync_copy(zbuf, accd.at[pl.ds(TILES * RPT + z * ZCH, ZCH)])

    pltpu.sync_copy(t_hbm, tv)
    plsc.subcore_barrier()

    cN = c * N_NODES
    col0 = c * HALF
    ebase = s * EPT

    def issue_sd(j, gx, dx, sm):
        base = ebase + j * CH
        pltpu.async_copy(sd_hbm.at[pl.ds(base, CH)], gx, sm)
        pltpu.async_copy(sd_hbm.at[pl.ds(N_EDGES + base, CH)], dx, sm)

    def wait_sd(gx, dx, sm):
        pltpu.make_async_copy(sd_hbm.at[pl.ds(0, CH)], gx, sm).wait()
        pltpu.make_async_copy(sd_hbm.at[pl.ds(0, CH)], dx, sm).wait()

    def issue_ge(j, gx, hr, er, gs, es):
        base = ebase + j * CH
        pltpu.async_copy(h_hbm.at[gx], hr, gs)
        pltpu.async_copy(e_hbm.at[pl.ds(c * N_EDGES + base, CH)], er, es)

    def wait_ge(gx, hr, er, gs, es):
        pltpu.make_async_copy(h_hbm.at[gx], hr, gs).wait()
        pltpu.make_async_copy(e_hbm.at[pl.ds(0, CH)], er, es).wait()

    def compute_scatter(hr, er, dx):
        tvec = tv[...]

        @plsc.parallel_loop(0, CH, unroll=4)
        def row(r):
            for v in range(HALF // 16):
                sl = pl.ds(v * 16, 16)
                m = jnp.maximum(hr[r, pl.ds(col0 + v * 16, 16)]
                                + er[r, sl], 0.0) + EPS
                w = jnp.exp(tvec * m)
                mwrows[r, sl] = m * w
                wrows[r, sl] = w
        d1 = pltpu.async_copy(mwrows, accn.at[dx], scS, add=True)
        d2 = pltpu.async_copy(wrows, accd.at[dx], scS, add=True)
        d1.wait()
        d2.wait()

    issue_sd(0, gidxA, didxA, ssA)
    wait_sd(gidxA, didxA, ssA)
    issue_ge(0, gidxA, hrowsA, erowsA, gsA, esA)
    issue_sd(1, gidxB, didxB, ssB)

    @pl.loop(0, NCHUNK // 2 - 1)
    def piter(k):
        j = 2 * k
        wait_sd(gidxB, didxB, ssB)
        issue_ge(j + 1, gidxB, hrowsB, erowsB, gsB, esB)
        wait_ge(gidxA, hrowsA, erowsA, gsA, esA)
        compute_scatter(hrowsA, erowsA, didxA)
        issue_sd(j + 2, gidxA, didxA, ssA)
        wait_ge(gidxB, hrowsB, erowsB, gsB, esB)
        compute_scatter(hrowsB, erowsB, didxB)
        issue_sd(j + 3, gidxB, didxB, ssB)
        wait_sd(gidxA, didxA, ssA)
        issue_ge(j + 2, gidxA, hrowsA, erowsA, gsA, esA)

    wait_sd(gidxB, didxB, ssB)
    issue_ge(NCHUNK - 1, gidxB, hrowsB, erowsB, gsB, esB)
    wait_ge(gidxA, hrowsA, erowsA, gsA, esA)
    compute_scatter(hrowsA, erowsA, didxA)
    wait_ge(gidxB, hrowsB, erowsB, gsB, esB)
    compute_scatter(hrowsB, erowsB, didxB)

    plsc.subcore_barrier()

    def wcopy(z, carry):
        r = row0 + z * ZCH
        pltpu.sync_copy(accn.at[pl.ds(r, ZCH)], zbuf)
        pltpu.sync_copy(zbuf, num_hbm.at[pl.ds(cN + r, ZCH)])
        pltpu.sync_copy(accd.at[pl.ds(r, ZCH)], zbuf)
        pltpu.sync_copy(zbuf, den_hbm.at[pl.ds(cN + r, ZCH)])
        return carry
    lax.fori_loop(0, RPT // ZCH, wcopy, 0)

    @pl.when(s == TILES - 1)
    def _write_tail():
        for z in range(2):
            r = TILES * RPT + z * ZCH
            pltpu.sync_copy(accn.at[pl.ds(r, ZCH)], zbuf)
            pltpu.sync_copy(zbuf, num_hbm.at[pl.ds(cN + r, ZCH)])
            pltpu.sync_copy(accd.at[pl.ds(r, ZCH)], zbuf)
            pltpu.sync_copy(zbuf, den_hbm.at[pl.ds(cN + r, ZCH)])


def _edge_pass(h, e2, sd_flat, t_i):
    num_flat, den_flat = _edge_kernel(
        h,
        e2.reshape(2 * N_EDGES, HALF),
        sd_flat,
        jnp.full((16,), t_i, jnp.float32))
    return (num_flat.reshape(2, N_NODES, HALF),
            den_flat.reshape(2, N_NODES, HALF))


def kernel(x, edge_attr, aW1, ab1, aW2, ab2, bW1, bb1, bW2, bb2, t,
           cW1, cb1, cg, cbe, cW2, cb2, l1W, l1b, l2W, l2b, edge_index, batch):
    sd_flat = edge_index.reshape(-1)
    h = _node_mlp(x, aW1, ab1, aW2, ab2)
    e2 = _edge_mlp(edge_attr, bW1, bb1, bW2, bb2)
    for i in range(N_LAYERS):
        num2, den2 = _edge_pass(h, e2, sd_flat, t[i])
        h = _node_update(num2, den2, h, cW1[i], cb1[i], cg[i], cbe[i],
                         cW2[i], cb2[i], relu_after=(i < N_LAYERS - 1))
    return _pool_head(h, batch, l1W, l1b, l2W, l2b)


# speedup vs baseline: 3.2555x; 1.0324x over previous
import functools

import jax
import jax.numpy as jnp
from jax import lax
from jax.experimental import pallas as pl
from jax.experimental.pallas import tpu as pltpu
from jax.experimental.pallas import tpu_sc as plsc

N_NODES = 10000
N_EDGES = 320000
D_IN = 128
D_EDGE = 16
HID = 128
OUT = 16
N_LAYERS = 3
N_GRAPHS = 128
EPS = 1e-7
HALF = HID // 2


def _mlp2_split_body(x_ref, w1_ref, b1_ref, w2_ref, b2_ref, out_ref):
    h = jnp.dot(x_ref[...], w1_ref[...], preferred_element_type=jnp.float32)
    h = jnp.maximum(h + b1_ref[...], 0.0)
    h = jnp.dot(h, w2_ref[...], preferred_element_type=jnp.float32)
    h = jnp.maximum(h + b2_ref[...], 0.0)
    out_ref[0] = h[:, :HALF]
    out_ref[1] = h[:, HALF:]


def _mlp2_body(x_ref, w1_ref, b1_ref, w2_ref, b2_ref, out_ref):
    h = jnp.dot(x_ref[...], w1_ref[...], preferred_element_type=jnp.float32)
    h = jnp.maximum(h + b1_ref[...], 0.0)
    h = jnp.dot(h, w2_ref[...], preferred_element_type=jnp.float32)
    out_ref[...] = jnp.maximum(h + b2_ref[...], 0.0)


def _node_mlp(x, w1, b1, w2, b2):
    return pl.pallas_call(
        _mlp2_split_body,
        out_shape=jax.ShapeDtypeStruct((2, N_NODES, HALF), jnp.float32),
    )(x, w1, b1.reshape(1, -1), w2, b2.reshape(1, -1))


def _edge_mlp(ea, w1, b1, w2, b2):
    eb = 10000
    return pl.pallas_call(
        _mlp2_split_body,
        grid=(N_EDGES // eb,),
        in_specs=[
            pl.BlockSpec((eb, D_EDGE), lambda i: (i, 0)),
            pl.BlockSpec((D_EDGE, HID), lambda i: (0, 0)),
            pl.BlockSpec((1, HID), lambda i: (0, 0)),
            pl.BlockSpec((HID, HID), lambda i: (0, 0)),
            pl.BlockSpec((1, HID), lambda i: (0, 0)),
        ],
        out_specs=pl.BlockSpec((2, eb, HALF), lambda i: (0, i, 0)),
        out_shape=jax.ShapeDtypeStruct((2, N_EDGES, HALF), jnp.float32),
    )(ea, w1, b1.reshape(1, -1), w2, b2.reshape(1, -1))


def _node_update_body(relu_after, num_ref, den_ref, h_ref, w1_ref, b1_ref,
                      g_ref, be_ref, w2_ref, b2_ref, out_ref):
    numer = jnp.concatenate([num_ref[0], num_ref[1]], axis=1)
    denom = jnp.concatenate([den_ref[0], den_ref[1]], axis=1)
    h = jnp.concatenate([h_ref[0], h_ref[1]], axis=1)
    out = numer / (denom + 1e-16) + h
    p = jnp.dot(out, w1_ref[...], preferred_element_type=jnp.float32) + b1_ref[...]
    mean = jnp.mean(p, axis=0, keepdims=True)
    var = jnp.mean((p - mean) ** 2, axis=0, keepdims=True)
    p = (p - mean) / jnp.sqrt(var + 1e-5) * g_ref[...] + be_ref[...]
    p = jnp.maximum(p, 0.0)
    r = jnp.dot(p, w2_ref[...], preferred_element_type=jnp.float32) + b2_ref[...]
    if relu_after:
        r = jnp.maximum(r, 0.0)
    out_ref[0] = r[:, :HALF]
    out_ref[1] = r[:, HALF:]


def _node_update(num2, den2, h2, w1, b1, g, be, w2, b2, relu_after):
    return pl.pallas_call(
        functools.partial(_node_update_body, relu_after),
        out_shape=jax.ShapeDtypeStruct((2, N_NODES, HALF), jnp.float32),
    )(num2, den2, h2, w1, b1.reshape(1, -1), g.reshape(1, -1),
      be.reshape(1, -1), w2, b2.reshape(1, -1))


def _pool_body(h_ref, batch_ref, l1w_ref, l1b_ref, l2w_ref, l2b_ref, out_ref):
    h = jnp.concatenate([h_ref[0], h_ref[1]], axis=1)
    gids = lax.broadcasted_iota(jnp.int32, (N_GRAPHS, N_NODES), 0)
    onehot = (batch_ref[...] == gids).astype(jnp.float32)
    pooled = jnp.dot(onehot, h, preferred_element_type=jnp.float32)
    z = jnp.dot(pooled, l1w_ref[...], preferred_element_type=jnp.float32)
    z = jnp.maximum(z + l1b_ref[...], 0.0)
    out_ref[...] = jnp.dot(z, l2w_ref[...], preferred_element_type=jnp.float32) + l2b_ref[...]


def _pool_head(h, batch, l1w, l1b, l2w, l2b):
    return pl.pallas_call(
        _pool_body,
        out_shape=jax.ShapeDtypeStruct((N_GRAPHS, OUT), jnp.float32),
    )(h, batch.reshape(1, -1), l1w, l1b.reshape(1, -1), l2w, l2b.reshape(1, -1))


TILES = 16
EPT = N_EDGES // TILES
CH = 40
NCHUNK = EPT // CH
RPT = 624
ZCH = 8

_mesh = plsc.VectorSubcoreMesh(core_axis_name="c", subcore_axis_name="s",
                               num_cores=2)


@functools.partial(
    pl.kernel,
    out_type=(jax.ShapeDtypeStruct((2 * N_NODES, HALF), jnp.float32),
              jax.ShapeDtypeStruct((2 * N_NODES, HALF), jnp.float32)),
    mesh=_mesh,
    compiler_params=pltpu.CompilerParams(use_tc_tiling_on_sc=False),
    scratch_types=(
        pltpu.VMEM((CH,), jnp.int32),
        pltpu.VMEM((CH,), jnp.int32),
        pltpu.VMEM((CH,), jnp.int32),
        pltpu.VMEM((CH,), jnp.int32),
        pltpu.VMEM((CH, HALF), jnp.float32),
        pltpu.VMEM((CH, HALF), jnp.float32),
        pltpu.VMEM((CH, HALF), jnp.float32),
        pltpu.VMEM((CH, HALF), jnp.float32),
        pltpu.VMEM((CH, HALF), jnp.float32),
        pltpu.VMEM((CH, HALF), jnp.float32),
        pltpu.VMEM((ZCH, HALF), jnp.float32),
        pltpu.VMEM((16,), jnp.float32),
        pltpu.VMEM_SHARED((N_NODES, HALF), jnp.float32),
        pltpu.VMEM_SHARED((N_NODES, HALF), jnp.float32),
        pltpu.SemaphoreType.DMA,
        pltpu.SemaphoreType.DMA,
        pltpu.SemaphoreType.DMA,
        pltpu.SemaphoreType.DMA,
        pltpu.SemaphoreType.DMA,
        pltpu.SemaphoreType.DMA,
        pltpu.SemaphoreType.DMA,
    ),
)
def _edge_kernel(h_hbm, e_hbm, sd_hbm, t_hbm, num_hbm, den_hbm,
                 gidxA, gidxB, didxA, didxB, hrowsA, hrowsB, erowsA, erowsB,
                 mwrows, wrows, zbuf, tv, accn, accd,
                 ssA, ssB, gsA, gsB, esA, esB, scS):
    c = lax.axis_index("c")
    s = lax.axis_index("s")

    def zfill(r, carry):
        for v in range(HALF // 16):
            zbuf[r, pl.ds(v * 16, 16)] = jnp.zeros((16,), jnp.float32)
        return carry
    lax.fori_loop(0, ZCH, zfill, 0)
    row0 = s * RPT

    def zcopy(z, carry):
        pltpu.sync_copy(zbuf, accn.at[pl.ds(row0 + z * ZCH, ZCH)])
        pltpu.sync_copy(zbuf, accd.at[pl.ds(row0 + z * ZCH, ZCH)])
        return carry
    lax.fori_loop(0, RPT // ZCH, zcopy, 0)

    @pl.when(s == TILES - 1)
    def _zero_tail():
        for z in range(2):
            pltpu.sync_copy(zbuf, accn.at[pl.ds(TILES * RPT + z * ZCH, ZCH)])
            pltpu.sync_copy(zbuf, accd.at[pl.ds(TILES * RPT + z * ZCH, ZCH)])

    pltpu.sync_copy(t_hbm, tv)
    plsc.subcore_barrier()

    cN = c * N_NODES
    col0 = c * HALF
    ebase = s * EPT

    def issue_sd(j, gx, dx, sm):
        base = ebase + j * CH
        pltpu.async_copy(sd_hbm.at[pl.ds(base, CH)], gx, sm)
        pltpu.async_copy(sd_hbm.at[pl.ds(N_EDGES + base, CH)], dx, sm)

    def wait_sd(gx, dx, sm):
        pltpu.make_async_copy(sd_hbm.at[pl.ds(0, CH)], gx, sm).wait()
        pltpu.make_async_copy(sd_hbm.at[pl.ds(0, CH)], dx, sm).wait()

    def issue_ge(j, gx, hr, er, gs, es):
        base = ebase + j * CH
        pltpu.async_copy(h_hbm.at[c].at[gx], hr, gs)
        pltpu.async_copy(e_hbm.at[pl.ds(c * N_EDGES + base, CH)], er, es)

    def wait_ge(gx, hr, er, gs, es):
        pltpu.make_async_copy(h_hbm.at[c].at[gx], hr, gs).wait()
        pltpu.make_async_copy(e_hbm.at[pl.ds(0, CH)], er, es).wait()

    def compute_scatter(hr, er, dx):
        tvec = tv[...]

        @plsc.parallel_loop(0, CH, unroll=4)
        def row(r):
            for v in range(HALF // 16):
                sl = pl.ds(v * 16, 16)
                m = jnp.maximum(hr[r, sl] + er[r, sl], 0.0) + EPS
                w = jnp.exp(tvec * m)
                mwrows[r, sl] = m * w
                wrows[r, sl] = w
        d1 = pltpu.async_copy(mwrows, accn.at[dx], scS, add=True)
        d2 = pltpu.async_copy(wrows, accd.at[dx], scS, add=True)
        d1.wait()
        d2.wait()

    issue_sd(0, gidxA, didxA, ssA)
    wait_sd(gidxA, didxA, ssA)
    issue_ge(0, gidxA, hrowsA, erowsA, gsA, esA)
    issue_sd(1, gidxB, didxB, ssB)

    @pl.loop(0, NCHUNK // 2 - 1)
    def piter(k):
        j = 2 * k
        wait_sd(gidxB, didxB, ssB)
        issue_ge(j + 1, gidxB, hrowsB, erowsB, gsB, esB)
        wait_ge(gidxA, hrowsA, erowsA, gsA, esA)
        compute_scatter(hrowsA, erowsA, didxA)
        issue_sd(j + 2, gidxA, didxA, ssA)
        wait_ge(gidxB, hrowsB, erowsB, gsB, esB)
        compute_scatter(hrowsB, erowsB, didxB)
        issue_sd(j + 3, gidxB, didxB, ssB)
        wait_sd(gidxA, didxA, ssA)
        issue_ge(j + 2, gidxA, hrowsA, erowsA, gsA, esA)

    wait_sd(gidxB, didxB, ssB)
    issue_ge(NCHUNK - 1, gidxB, hrowsB, erowsB, gsB, esB)
    wait_ge(gidxA, hrowsA, erowsA, gsA, esA)
    compute_scatter(hrowsA, erowsA, didxA)
    wait_ge(gidxB, hrowsB, erowsB, gsB, esB)
    compute_scatter(hrowsB, erowsB, didxB)

    plsc.subcore_barrier()

    def wcopy(z, carry):
        r = row0 + z * ZCH
        pltpu.sync_copy(accn.at[pl.ds(r, ZCH)], zbuf)
        pltpu.sync_copy(zbuf, num_hbm.at[pl.ds(cN + r, ZCH)])
        pltpu.sync_copy(accd.at[pl.ds(r, ZCH)], zbuf)
        pltpu.sync_copy(zbuf, den_hbm.at[pl.ds(cN + r, ZCH)])
        return carry
    lax.fori_loop(0, RPT // ZCH, wcopy, 0)

    @pl.when(s == TILES - 1)
    def _write_tail():
        for z in range(2):
            r = TILES * RPT + z * ZCH
            pltpu.sync_copy(accn.at[pl.ds(r, ZCH)], zbuf)
            pltpu.sync_copy(zbuf, num_hbm.at[pl.ds(cN + r, ZCH)])
            pltpu.sync_copy(accd.at[pl.ds(r, ZCH)], zbuf)
            pltpu.sync_copy(zbuf, den_hbm.at[pl.ds(cN + r, ZCH)])


def _edge_pass(h2, e2, sd_flat, t_i):
    num_flat, den_flat = _edge_kernel(
        h2,
        e2.reshape(2 * N_EDGES, HALF),
        sd_flat,
        jnp.full((16,), t_i, jnp.float32))
    return (num_flat.reshape(2, N_NODES, HALF),
            den_flat.reshape(2, N_NODES, HALF))


def kernel(x, edge_attr, aW1, ab1, aW2, ab2, bW1, bb1, bW2, bb2, t,
           cW1, cb1, cg, cbe, cW2, cb2, l1W, l1b, l2W, l2b, edge_index, batch):
    sd_flat = edge_index.reshape(-1)
    h2 = _node_mlp(x, aW1, ab1, aW2, ab2)
    e2 = _edge_mlp(edge_attr, bW1, bb1, bW2, bb2)
    for i in range(N_LAYERS):
        num2, den2 = _edge_pass(h2, e2, sd_flat, t[i])
        h2 = _node_update(num2, den2, h2, cW1[i], cb1[i], cg[i], cbe[i],
                          cW2[i], cb2[i], relu_after=(i < N_LAYERS - 1))
    return _pool_head(h2, batch, l1W, l1b, l2W, l2b)
